# baseline (speedup 1.0000x reference)
import functools

import jax
import jax.numpy as jnp
from jax import lax
from jax.experimental import pallas as pl
from jax.experimental.pallas import tpu as pltpu
from jax.experimental.pallas import tpu_sc as plsc

_B = 8
_L = 128
_NPER = 1024
_D = 512
_DG = 256
_NW = 32
_ROWS_PER_W = (_B * _L) // _NW


def _emb_gather_body(table_hbm, idx_hbm, out_hbm, idx_v, rows_v, sem):
    wid = lax.axis_index("s") * 2 + lax.axis_index("c")
    base = wid * _ROWS_PER_W
    pltpu.sync_copy(idx_hbm.at[pl.ds(base, _ROWS_PER_W)], idx_v)
    pltpu.async_copy(table_hbm.at[idx_v], rows_v, sem).wait()
    pltpu.sync_copy(rows_v, out_hbm.at[pl.ds(base, _ROWS_PER_W)])


@functools.lru_cache(maxsize=None)
def _make_emb_gather():
    return pl.kernel(
        _emb_gather_body,
        jax.ShapeDtypeStruct((_B * _L, _D), jnp.float32),
        mesh=plsc.VectorSubcoreMesh(core_axis_name="c", subcore_axis_name="s"),
        scratch_types=[
            pltpu.VMEM((_ROWS_PER_W,), jnp.int32),
            pltpu.VMEM((_ROWS_PER_W, _D), jnp.float32),
            pltpu.SemaphoreType.DMA,
        ],
    )


def _states_body(x_ref, wg_ref, bg_ref, ws_ref, bs_ref, st_ref):
    h1 = jnp.tanh(
        jnp.dot(x_ref[...], wg_ref[...], preferred_element_type=jnp.float32)
        + bg_ref[...]
    )
    st = jnp.dot(h1.astype(jnp.bfloat16), ws_ref[...],
                 preferred_element_type=jnp.float32) + bs_ref[...]
    st_ref[...] = st.astype(jnp.bfloat16)


def _states(x_bf, wg, bg, ws, bs):
    return pl.pallas_call(
        _states_body,
        grid=(_B,),
        in_specs=[
            pl.BlockSpec((_NPER, _D), lambda b: (b, 0)),
            pl.BlockSpec((_D, _DG), lambda b: (0, 0)),
            pl.BlockSpec((1, _DG), lambda b: (0, 0)),
            pl.BlockSpec((_DG, _D), lambda b: (0, 0)),
            pl.BlockSpec((1, _D), lambda b: (0, 0)),
        ],
        out_specs=pl.BlockSpec((_NPER, _D), lambda b: (b, 0)),
        out_shape=jax.ShapeDtypeStruct((_B * _NPER, _D), jnp.bfloat16),
    )(x_bf, wg, bg, ws, bs)


def _sigmoid(v):
    return 1.0 / (1.0 + jnp.exp(-v))


def _gru_body(e_ref, wx_ref, wh_ref, bx_ref, bh_ref, m_ref, cs_ref, gx_ref):
    gx_ref[...] = (
        jnp.dot(e_ref[...].astype(jnp.bfloat16), wx_ref[...],
                preferred_element_type=jnp.float32)
        + bx_ref[...]
    )

    def step(t, carry):
        h, acc = carry
        gxt = gx_ref[pl.ds(t * _B, _B), :]
        gh = jnp.dot(h.astype(jnp.bfloat16), wh_ref[...],
                     preferred_element_type=jnp.float32) + bh_ref[...]
        z = _sigmoid(gxt[:, :_D] + gh[:, :_D])
        r = _sigmoid(gxt[:, _D:2 * _D] + gh[:, _D:2 * _D])
        c = jnp.tanh(gxt[:, 2 * _D:] + r * gh[:, 2 * _D:])
        h2 = z * h + (1.0 - z) * c
        acc2 = acc + h2 * m_ref[t]
        return (h2, acc2)

    zeros = jnp.zeros((_B, _D), jnp.float32)
    _, acc = lax.fori_loop(0, _L, step, (zeros, zeros), unroll=4)
    cs_ref[...] = acc / jnp.sqrt(jnp.float32(_D))


def _gru(e, wx, wh, bx, bh, m):
    return pl.pallas_call(
        _gru_body,
        out_shape=jax.ShapeDtypeStruct((_B, _D), jnp.float32),
        scratch_shapes=[pltpu.VMEM((_B * _L, 3 * _D), jnp.float32)],
    )(e, wx, wh, bx, bh, m)


def _score_body(keys_ref, st_ref, cs_ref, choice_ref, misc_ref, acc_ref):
    b = pl.program_id(0)
    st = st_ref[...].astype(jnp.float32)
    s = jnp.sum(st * cs_ref[0], axis=1, keepdims=True)
    smax = jnp.max(s)
    e = jnp.exp(s - smax)
    p = e / jnp.sum(e)
    choice_ref[...] = p

    key_b = keys_ref[b]
    iota = lax.broadcasted_iota(jnp.int32, (_NPER, 1), 0)
    p_at = jnp.sum(jnp.where(iota == key_b, p, 0.0))
    loss_b = -jnp.log(1e-6 + p_at)
    pmax = jnp.max(p)
    arg = jnp.min(jnp.where(p >= pmax, iota, jnp.int32(_NPER)))
    corr = jnp.where(arg == key_b, jnp.int32(1), jnp.int32(0))

    li = lax.broadcasted_iota(jnp.int32, (1, 128), 1)
    misc_ref[...] = jnp.where(li == 0, loss_b, 0.0).reshape(1, 1, 128)

    @pl.when(b == 0)
    def _():
        acc_ref[...] = jnp.zeros((1, 128), jnp.int32)

    acc_ref[...] += jnp.where(li == 0, corr, jnp.int32(0))


def _scores(keys, st, cs):
    return pl.pallas_call(
        _score_body,
        grid=(_B,),
        in_specs=[
            pl.BlockSpec(memory_space=pltpu.SMEM),
            pl.BlockSpec((_NPER, _D), lambda b: (b, 0)),
            pl.BlockSpec((1, 1, _D), lambda b: (b, 0, 0)),
        ],
        out_specs=[
            pl.BlockSpec((_NPER, 1), lambda b: (b, 0)),
            pl.BlockSpec((1, 1, 128), lambda b: (b, 0, 0)),
            pl.BlockSpec((1, 128), lambda b: (0, 0)),
        ],
        out_shape=[
            jax.ShapeDtypeStruct((_B * _NPER, 1), jnp.float32),
            jax.ShapeDtypeStruct((_B, 1, 128), jnp.float32),
            jax.ShapeDtypeStruct((1, 128), jnp.int32),
        ],
    )(keys, st, cs.reshape(_B, 1, _D))


def kernel(x, comment_indexes, comment_masks, comment_keys, W_ggnn, b_ggnn,
           W_scale, b_scale, embed, Wx, Wh, bx, bh):
    idx = comment_indexes.astype(jnp.int32).T.reshape(-1)
    m = comment_masks.astype(jnp.float32).T.reshape(_L, _B, 1)

    e = _make_emb_gather()(embed, idx)
    st = _states(x.astype(jnp.bfloat16), W_ggnn.astype(jnp.bfloat16),
                 b_ggnn.reshape(1, -1), W_scale.astype(jnp.bfloat16),
                 b_scale.reshape(1, -1))
    cs = _gru(e, Wx.astype(jnp.bfloat16), Wh.astype(jnp.bfloat16),
              bx.reshape(1, -1), bh.reshape(1, -1), m)
    choice2, misc, accv = _scores(comment_keys.astype(jnp.int32), st, cs)

    choice = choice2.reshape(-1)
    loss_at = misc[:, 0, 0]
    acc = accv[0, 0]
    return (choice, loss_at, acc)


# speedup vs baseline: 1.1879x; 1.1879x over previous
import functools

import jax
import jax.numpy as jnp
from jax import lax
from jax.experimental import pallas as pl
from jax.experimental.pallas import tpu as pltpu
from jax.experimental.pallas import tpu_sc as plsc

_B = 8
_L = 128
_NPER = 1024
_D = 512
_DG = 256
_NW = 32
_ROWS_PER_W = (_B * _L) // _NW


def _emb_gather_body(table_hbm, idx_hbm, out_hbm, idx_v, rows_v, sem):
    wid = lax.axis_index("s") * 2 + lax.axis_index("c")
    base = wid * _ROWS_PER_W
    pltpu.sync_copy(idx_hbm.at[pl.ds(base, _ROWS_PER_W)], idx_v)
    pltpu.async_copy(table_hbm.at[idx_v], rows_v, sem).wait()
    pltpu.sync_copy(rows_v, out_hbm.at[pl.ds(base, _ROWS_PER_W)])


@functools.lru_cache(maxsize=None)
def _make_emb_gather():
    return pl.kernel(
        _emb_gather_body,
        jax.ShapeDtypeStruct((_B * _L, _D), jnp.float32),
        mesh=plsc.VectorSubcoreMesh(core_axis_name="c", subcore_axis_name="s"),
        scratch_types=[
            pltpu.VMEM((_ROWS_PER_W,), jnp.int32),
            pltpu.VMEM((_ROWS_PER_W, _D), jnp.float32),
            pltpu.SemaphoreType.DMA,
        ],
    )


def _states_body(x_ref, wg_ref, bg_ref, ws_ref, bs_ref, st_ref):
    h1t = jnp.tanh(
        lax.dot_general(wg_ref[...], x_ref[...], (((0,), (1,)), ((), ())),
                        preferred_element_type=jnp.float32)
        + bg_ref[...]
    )
    stt = lax.dot_general(ws_ref[...], h1t, (((0,), (0,)), ((), ())),
                          preferred_element_type=jnp.float32) + bs_ref[...]
    st_ref[...] = stt.astype(jnp.bfloat16)


def _states(x, wg, bg_col, ws, bs_col):
    return pl.pallas_call(
        _states_body,
        grid=(_B,),
        in_specs=[
            pl.BlockSpec((_NPER, _D), lambda b: (b, 0)),
            pl.BlockSpec((_D, _DG), lambda b: (0, 0)),
            pl.BlockSpec((_DG, 1), lambda b: (0, 0)),
            pl.BlockSpec((_DG, _D), lambda b: (0, 0)),
            pl.BlockSpec((_D, 1), lambda b: (0, 0)),
        ],
        out_specs=pl.BlockSpec((_D, _NPER), lambda b: (0, b)),
        out_shape=jax.ShapeDtypeStruct((_D, _B * _NPER), jnp.bfloat16),
    )(x, wg, bg_col, ws, bs_col)


def _sigmoid(v):
    return 1.0 / (1.0 + jnp.exp(-v))


def _gru_body(e_ref, wx_ref, wh_ref, bx_ref, bh_ref, m_ref, cs_ref, gx_ref):
    gx_ref[...] = (
        jnp.dot(e_ref[...], wx_ref[...], preferred_element_type=jnp.float32)
        + bx_ref[...]
    )

    def step(t, carry):
        h, acc = carry
        gxt = gx_ref[pl.ds(t * _B, _B), :]
        gh = jnp.dot(h, wh_ref[...], preferred_element_type=jnp.float32) + bh_ref[...]
        z = _sigmoid(gxt[:, :_D] + gh[:, :_D])
        r = _sigmoid(gxt[:, _D:2 * _D] + gh[:, _D:2 * _D])
        c = jnp.tanh(gxt[:, 2 * _D:] + r * gh[:, 2 * _D:])
        h2 = z * h + (1.0 - z) * c
        acc2 = acc + h2 * m_ref[t]
        return (h2, acc2)

    zeros = jnp.zeros((_B, _D), jnp.float32)
    _, acc = lax.fori_loop(0, _L, step, (zeros, zeros), unroll=4)
    cs_ref[...] = acc / jnp.sqrt(jnp.float32(_D))


def _gru(e, wx, wh, bx, bh, m):
    return pl.pallas_call(
        _gru_body,
        out_shape=jax.ShapeDtypeStruct((_B, _D), jnp.float32),
        scratch_shapes=[pltpu.VMEM((_B * _L, 3 * _D), jnp.float32)],
    )(e, wx, wh, bx, bh, m)


def _score_body(keys_ref, st_ref, cs_ref, choice_ref, misc_ref, acc_ref):
    b = pl.program_id(0)
    s = lax.dot_general(cs_ref[0].astype(jnp.bfloat16), st_ref[...],
                        (((1,), (0,)), ((), ())),
                        preferred_element_type=jnp.float32)
    smax = jnp.max(s)
    e = jnp.exp(s - smax)
    p = e / jnp.sum(e)
    choice_ref[...] = p.reshape(1, 1, _NPER)

    key_b = keys_ref[b]
    iota = lax.broadcasted_iota(jnp.int32, (1, _NPER), 1)
    p_at = jnp.sum(jnp.where(iota == key_b, p, 0.0))
    loss_b = -jnp.log(1e-6 + p_at)
    pmax = jnp.max(p)
    arg = jnp.min(jnp.where(p >= pmax, iota, jnp.int32(_NPER)))
    corr = jnp.where(arg == key_b, jnp.int32(1), jnp.int32(0))

    li = lax.broadcasted_iota(jnp.int32, (1, 128), 1)
    misc_ref[...] = jnp.where(li == 0, loss_b, 0.0).reshape(1, 1, 128)

    @pl.when(b == 0)
    def _():
        acc_ref[...] = jnp.zeros((1, 128), jnp.int32)

    acc_ref[...] += jnp.where(li == 0, corr, jnp.int32(0))


def _scores(keys, st, cs):
    return pl.pallas_call(
        _score_body,
        grid=(_B,),
        in_specs=[
            pl.BlockSpec(memory_space=pltpu.SMEM),
            pl.BlockSpec((_D, _NPER), lambda b: (0, b)),
            pl.BlockSpec((1, 1, _D), lambda b: (b, 0, 0)),
        ],
        out_specs=[
            pl.BlockSpec((1, 1, _NPER), lambda b: (b, 0, 0)),
            pl.BlockSpec((1, 1, 128), lambda b: (b, 0, 0)),
            pl.BlockSpec((1, 128), lambda b: (0, 0)),
        ],
        out_shape=[
            jax.ShapeDtypeStruct((_B, 1, _NPER), jnp.float32),
            jax.ShapeDtypeStruct((_B, 1, 128), jnp.float32),
            jax.ShapeDtypeStruct((1, 128), jnp.int32),
        ],
    )(keys, st, cs.reshape(_B, 1, _D))


def kernel(x, comment_indexes, comment_masks, comment_keys, W_ggnn, b_ggnn,
           W_scale, b_scale, embed, Wx, Wh, bx, bh):
    idx = comment_indexes.astype(jnp.int32).T.reshape(-1)
    m = comment_masks.astype(jnp.float32).T.reshape(_L, _B, 1)

    e = _make_emb_gather()(embed, idx)
    st = _states(x, W_ggnn, b_ggnn.reshape(-1, 1), W_scale,
                 b_scale.reshape(-1, 1))
    cs = _gru(e, Wx, Wh, bx.reshape(1, -1), bh.reshape(1, -1), m)
    choice3, misc, accv = _scores(comment_keys.astype(jnp.int32), st, cs)

    choice = choice3.reshape(-1)
    loss_at = misc[:, 0, 0]
    acc = accv[0, 0]
    return (choice, loss_at, acc)


# speedup vs baseline: 1.2084x; 1.0173x over previous
import functools

import jax
import jax.numpy as jnp
from jax import lax
from jax.experimental import pallas as pl
from jax.experimental.pallas import tpu as pltpu
from jax.experimental.pallas import tpu_sc as plsc

_B = 8
_L = 128
_NPER = 1024
_D = 512
_DG = 256
_NW = 32
_ROWS_PER_W = (_B * _L) // _NW


def _emb_gather_body(table_hbm, idx_hbm, out_hbm, idx_v, rows_v, sem):
    wid = lax.axis_index("s") * 2 + lax.axis_index("c")
    base = wid * _ROWS_PER_W
    pltpu.sync_copy(idx_hbm.at[pl.ds(base, _ROWS_PER_W)], idx_v)
    pltpu.async_copy(table_hbm.at[idx_v], rows_v, sem).wait()
    pltpu.sync_copy(rows_v, out_hbm.at[pl.ds(base, _ROWS_PER_W)])


@functools.lru_cache(maxsize=None)
def _make_emb_gather():
    return pl.kernel(
        _emb_gather_body,
        jax.ShapeDtypeStruct((_B * _L, _D), jnp.float32),
        mesh=plsc.VectorSubcoreMesh(core_axis_name="c", subcore_axis_name="s"),
        scratch_types=[
            pltpu.VMEM((_ROWS_PER_W,), jnp.int32),
            pltpu.VMEM((_ROWS_PER_W, _D), jnp.float32),
            pltpu.SemaphoreType.DMA,
        ],
    )


def _states_body(x_ref, wg_ref, bg_ref, ws_ref, bs_ref, st_ref):
    h1t = jnp.tanh(
        lax.dot_general(wg_ref[...], x_ref[...], (((0,), (1,)), ((), ())),
                        preferred_element_type=jnp.float32)
        + bg_ref[...]
    )
    stt = lax.dot_general(ws_ref[...], h1t, (((0,), (0,)), ((), ())),
                          preferred_element_type=jnp.float32) + bs_ref[...]
    st_ref[...] = stt.astype(jnp.bfloat16)


def _states(x, wg, bg_col, ws, bs_col):
    return pl.pallas_call(
        _states_body,
        grid=(_B,),
        in_specs=[
            pl.BlockSpec((_NPER, _D), lambda b: (b, 0)),
            pl.BlockSpec((_D, _DG), lambda b: (0, 0)),
            pl.BlockSpec((_DG, 1), lambda b: (0, 0)),
            pl.BlockSpec((_DG, _D), lambda b: (0, 0)),
            pl.BlockSpec((_D, 1), lambda b: (0, 0)),
        ],
        out_specs=pl.BlockSpec((_D, _NPER), lambda b: (0, b)),
        out_shape=jax.ShapeDtypeStruct((_D, _B * _NPER), jnp.bfloat16),
    )(x, wg, bg_col, ws, bs_col)


def _sigmoid(v):
    return 0.5 * jnp.tanh(0.5 * v) + 0.5


def _gru_body(e_ref, wx_ref, wh_ref, bx_ref, bh_ref, m_ref, cs_ref,
              gx_ref, whzr_ref, whh_ref):
    whzr_ref[...] = wh_ref[:, :2 * _D].astype(jnp.bfloat16)
    whh_ref[...] = wh_ref[:, 2 * _D:].astype(jnp.bfloat16)
    gx_ref[...] = (
        jnp.dot(e_ref[...], wx_ref[...], preferred_element_type=jnp.float32)
        + bx_ref[...]
    )

    def step(t, carry):
        h, acc = carry
        gxt = gx_ref[pl.ds(t * _B, _B), :]
        hb = h.astype(jnp.bfloat16)
        gh_zr = jnp.dot(hb, whzr_ref[...], preferred_element_type=jnp.float32)
        gh_h = jnp.dot(hb, whh_ref[...], preferred_element_type=jnp.float32)
        z = _sigmoid(gxt[:, :_D] + gh_zr[:, :_D] + bh_ref[:, :_D])
        r = _sigmoid(gxt[:, _D:2 * _D] + gh_zr[:, _D:] + bh_ref[:, _D:2 * _D])
        c = jnp.tanh(gxt[:, 2 * _D:] + r * (gh_h + bh_ref[:, 2 * _D:]))
        h2 = z * h + (1.0 - z) * c
        acc2 = acc + h2 * m_ref[t]
        return (h2, acc2)

    zeros = jnp.zeros((_B, _D), jnp.float32)
    _, acc = lax.fori_loop(0, _L, step, (zeros, zeros), unroll=8)
    cs_ref[...] = (acc / jnp.sqrt(jnp.float32(_D))).reshape(_B, 1, _D)


def _gru(e, wx, wh, bx, bh, m):
    return pl.pallas_call(
        _gru_body,
        out_shape=jax.ShapeDtypeStruct((_B, 1, _D), jnp.float32),
        scratch_shapes=[
            pltpu.VMEM((_B * _L, 3 * _D), jnp.float32),
            pltpu.VMEM((_D, 2 * _D), jnp.bfloat16),
            pltpu.VMEM((_D, _D), jnp.bfloat16),
        ],
    )(e, wx, wh, bx, bh, m)


def _score_body(keys_ref, st_ref, cs_ref, choice_ref, misc_ref, acc_ref):
    b = pl.program_id(0)
    s = lax.dot_general(cs_ref[0].astype(jnp.bfloat16), st_ref[...],
                        (((1,), (0,)), ((), ())),
                        preferred_element_type=jnp.float32)
    smax = jnp.max(s)
    e = jnp.exp(s - smax)
    p = e / jnp.sum(e)
    choice_ref[...] = p.reshape(1, 1, _NPER)

    key_b = keys_ref[b]
    iota = lax.broadcasted_iota(jnp.int32, (1, _NPER), 1)
    p_at = jnp.sum(jnp.where(iota == key_b, p, 0.0))
    loss_b = -jnp.log(1e-6 + p_at)
    pmax = jnp.max(p)
    arg = jnp.min(jnp.where(p >= pmax, iota, jnp.int32(_NPER)))
    corr = jnp.where(arg == key_b, jnp.int32(1), jnp.int32(0))

    li = lax.broadcasted_iota(jnp.int32, (1, 128), 1)
    misc_ref[...] = jnp.where(li == 0, loss_b, 0.0).reshape(1, 1, 128)

    @pl.when(b == 0)
    def _():
        acc_ref[...] = jnp.zeros((1, 128), jnp.int32)

    acc_ref[...] += jnp.where(li == 0, corr, jnp.int32(0))


def _scores(keys, st, cs):
    return pl.pallas_call(
        _score_body,
        grid=(_B,),
        in_specs=[
            pl.BlockSpec(memory_space=pltpu.SMEM),
            pl.BlockSpec((_D, _NPER), lambda b: (0, b)),
            pl.BlockSpec((1, 1, _D), lambda b: (b, 0, 0)),
        ],
        out_specs=[
            pl.BlockSpec((1, 1, _NPER), lambda b: (b, 0, 0)),
            pl.BlockSpec((1, 1, 128), lambda b: (b, 0, 0)),
            pl.BlockSpec((1, 128), lambda b: (0, 0)),
        ],
        out_shape=[
            jax.ShapeDtypeStruct((_B, 1, _NPER), jnp.float32),
            jax.ShapeDtypeStruct((_B, 1, 128), jnp.float32),
            jax.ShapeDtypeStruct((1, 128), jnp.int32),
        ],
    )(keys, st, cs)


def kernel(x, comment_indexes, comment_masks, comment_keys, W_ggnn, b_ggnn,
           W_scale, b_scale, embed, Wx, Wh, bx, bh):
    idx = comment_indexes.astype(jnp.int32).T.reshape(-1)
    m = comment_masks.astype(jnp.float32).T.reshape(_L, _B, 1)

    e = _make_emb_gather()(embed, idx)
    st = _states(x, W_ggnn, b_ggnn.reshape(-1, 1), W_scale,
                 b_scale.reshape(-1, 1))
    cs = _gru(e, Wx, Wh, bx.reshape(1, -1), bh.reshape(1, -1), m)
    choice3, misc, accv = _scores(comment_keys.astype(jnp.int32), st, cs)

    choice = choice3.reshape(-1)
    loss_at = misc[:, 0, 0]
    acc = accv[0, 0]
    return (choice, loss_at, acc)


# speedup vs baseline: 1.3164x; 1.0894x over previous
import functools

import jax
import jax.numpy as jnp
from jax import lax
from jax.experimental import pallas as pl
from jax.experimental.pallas import tpu as pltpu
from jax.experimental.pallas import tpu_sc as plsc

_B = 8
_L = 128
_NPER = 1024
_D = 512
_DG = 256
_NW = 32
_ROWS_PER_W = (_B * _L) // _NW


def _emb_gather_body(table_hbm, idx_hbm, out_hbm, idx_v, rows_v, sem):
    wid = lax.axis_index("s") * 2 + lax.axis_index("c")
    base = wid * _ROWS_PER_W
    pltpu.sync_copy(idx_hbm.at[pl.ds(base, _ROWS_PER_W)], idx_v)
    pltpu.async_copy(table_hbm.at[idx_v], rows_v, sem).wait()
    pltpu.sync_copy(rows_v, out_hbm.at[pl.ds(base, _ROWS_PER_W)])


@functools.lru_cache(maxsize=None)
def _make_emb_gather():
    return pl.kernel(
        _emb_gather_body,
        jax.ShapeDtypeStruct((_B * _L, _D), jnp.float32),
        mesh=plsc.VectorSubcoreMesh(core_axis_name="c", subcore_axis_name="s"),
        scratch_types=[
            pltpu.VMEM((_ROWS_PER_W,), jnp.int32),
            pltpu.VMEM((_ROWS_PER_W, _D), jnp.float32),
            pltpu.SemaphoreType.DMA,
        ],
    )


def _sigmoid(v):
    return 0.5 * jnp.tanh(0.5 * v) + 0.5


def _gru_body(e_ref, wx_ref, wh_ref, bx_ref, bh_ref, m_ref, cs_ref,
              gx_ref, whzr_ref, whh_ref):
    whzr_ref[...] = wh_ref[:, :2 * _D].astype(jnp.bfloat16)
    whh_ref[...] = wh_ref[:, 2 * _D:].astype(jnp.bfloat16)
    gx_ref[...] = (
        jnp.dot(e_ref[...], wx_ref[...], preferred_element_type=jnp.float32)
        + bx_ref[...]
    )

    def step(t, carry):
        h, acc = carry
        gxt = gx_ref[pl.ds(t * _B, _B), :]
        hb = h.astype(jnp.bfloat16)
        gh_zr = jnp.dot(hb, whzr_ref[...], preferred_element_type=jnp.float32)
        gh_h = jnp.dot(hb, whh_ref[...], preferred_element_type=jnp.float32)
        z = _sigmoid(gxt[:, :_D] + gh_zr[:, :_D] + bh_ref[:, :_D])
        r = _sigmoid(gxt[:, _D:2 * _D] + gh_zr[:, _D:] + bh_ref[:, _D:2 * _D])
        c = jnp.tanh(gxt[:, 2 * _D:] + r * (gh_h + bh_ref[:, 2 * _D:]))
        h2 = z * h + (1.0 - z) * c
        acc2 = acc + h2 * m_ref[t]
        return (h2, acc2)

    zeros = jnp.zeros((_B, _D), jnp.float32)
    _, acc = lax.fori_loop(0, _L, step, (zeros, zeros), unroll=8)
    cs_ref[...] = (acc / jnp.sqrt(jnp.float32(_D))).reshape(_B, 1, _D)


def _gru(e, wx, wh, bx, bh, m):
    return pl.pallas_call(
        _gru_body,
        out_shape=jax.ShapeDtypeStruct((_B, 1, _D), jnp.float32),
        scratch_shapes=[
            pltpu.VMEM((_B * _L, 3 * _D), jnp.float32),
            pltpu.VMEM((_D, 2 * _D), jnp.bfloat16),
            pltpu.VMEM((_D, _D), jnp.bfloat16),
        ],
    )(e, wx, wh, bx, bh, m)


def _score_body(keys_ref, x_ref, wg_ref, bg_ref, ws_ref, bs_ref, cs_ref,
                choice_ref, misc_ref, acc_ref):
    b = pl.program_id(0)
    h1t = jnp.tanh(
        lax.dot_general(wg_ref[...], x_ref[...], (((0,), (1,)), ((), ())),
                        preferred_element_type=jnp.float32)
        + bg_ref[...]
    )
    stt = lax.dot_general(ws_ref[...], h1t, (((0,), (0,)), ((), ())),
                          preferred_element_type=jnp.float32) + bs_ref[...]
    s = lax.dot_general(cs_ref[0], stt, (((1,), (0,)), ((), ())),
                        preferred_element_type=jnp.float32)
    smax = jnp.max(s)
    e = jnp.exp(s - smax)
    p = e / jnp.sum(e)
    choice_ref[...] = p.reshape(1, 1, _NPER)

    key_b = keys_ref[b]
    iota = lax.broadcasted_iota(jnp.int32, (1, _NPER), 1)
    p_at = jnp.sum(jnp.where(iota == key_b, p, 0.0))
    loss_b = -jnp.log(1e-6 + p_at)
    pmax = jnp.max(p)
    arg = jnp.min(jnp.where(p >= pmax, iota, jnp.int32(_NPER)))
    corr = jnp.where(arg == key_b, jnp.int32(1), jnp.int32(0))

    li = lax.broadcasted_iota(jnp.int32, (1, 128), 1)
    misc_ref[...] = jnp.where(li == 0, loss_b, 0.0).reshape(1, 1, 128)

    @pl.when(b == 0)
    def _():
        acc_ref[...] = jnp.zeros((1, 128), jnp.int32)

    acc_ref[...] += jnp.where(li == 0, corr, jnp.int32(0))


def _scores(keys, x, wg, bg_col, ws, bs_col, cs):
    return pl.pallas_call(
        _score_body,
        grid=(_B,),
        in_specs=[
            pl.BlockSpec(memory_space=pltpu.SMEM),
            pl.BlockSpec((_NPER, _D), lambda b: (b, 0)),
            pl.BlockSpec((_D, _DG), lambda b: (0, 0)),
            pl.BlockSpec((_DG, 1), lambda b: (0, 0)),
            pl.BlockSpec((_DG, _D), lambda b: (0, 0)),
            pl.BlockSpec((_D, 1), lambda b: (0, 0)),
            pl.BlockSpec((1, 1, _D), lambda b: (b, 0, 0)),
        ],
        out_specs=[
            pl.BlockSpec((1, 1, _NPER), lambda b: (b, 0, 0)),
            pl.BlockSpec((1, 1, 128), lambda b: (b, 0, 0)),
            pl.BlockSpec((1, 128), lambda b: (0, 0)),
        ],
        out_shape=[
            jax.ShapeDtypeStruct((_B, 1, _NPER), jnp.float32),
            jax.ShapeDtypeStruct((_B, 1, 128), jnp.float32),
            jax.ShapeDtypeStruct((1, 128), jnp.int32),
        ],
    )(keys, x, wg, bg_col, ws, bs_col, cs)


def kernel(x, comment_indexes, comment_masks, comment_keys, W_ggnn, b_ggnn,
           W_scale, b_scale, embed, Wx, Wh, bx, bh):
    idx = comment_indexes.astype(jnp.int32).T.reshape(-1)
    m = comment_masks.astype(jnp.float32).T.reshape(_L, _B, 1)

    e = _make_emb_gather()(embed, idx)
    cs = _gru(e, Wx, Wh, bx.reshape(1, -1), bh.reshape(1, -1), m)
    choice3, misc, accv = _scores(
        comment_keys.astype(jnp.int32), x, W_ggnn, b_ggnn.reshape(-1, 1),
        W_scale, b_scale.reshape(-1, 1), cs)

    choice = choice3.reshape(-1)
    loss_at = misc[:, 0, 0]
    acc = accv[0, 0]
    return (choice, loss_at, acc)


# speedup vs baseline: 1.3198x; 1.0025x over previous
import functools

import jax
import jax.numpy as jnp
from jax import lax
from jax.experimental import pallas as pl
from jax.experimental.pallas import tpu as pltpu
from jax.experimental.pallas import tpu_sc as plsc

_B = 8
_L = 128
_NPER = 1024
_D = 512
_DG = 256
_NW = 32
_ROWS_PER_W = (_B * _L) // _NW


def _emb_gather_body(table_hbm, idx_hbm, out_hbm, idx_v, rows_v, sem):
    wid = lax.axis_index("s") * 2 + lax.axis_index("c")
    base = wid * _ROWS_PER_W
    pltpu.sync_copy(idx_hbm.at[pl.ds(base, _ROWS_PER_W)], idx_v)
    pltpu.async_copy(table_hbm.at[idx_v], rows_v, sem).wait()
    pltpu.sync_copy(rows_v, out_hbm.at[pl.ds(base, _ROWS_PER_W)])


@functools.lru_cache(maxsize=None)
def _make_emb_gather():
    return pl.kernel(
        _emb_gather_body,
        jax.ShapeDtypeStruct((_B * _L, _D), jnp.float32),
        mesh=plsc.VectorSubcoreMesh(core_axis_name="c", subcore_axis_name="s"),
        scratch_types=[
            pltpu.VMEM((_ROWS_PER_W,), jnp.int32),
            pltpu.VMEM((_ROWS_PER_W, _D), jnp.float32),
            pltpu.SemaphoreType.DMA,
        ],
    )


def _sigmoid(v):
    return 0.5 * jnp.tanh(0.5 * v) + 0.5


def _gru_body(e_ref, wx_ref, wh_ref, bx_ref, bh_ref, m_ref, cs_ref,
              gx_ref, whzr_ref, whh_ref):
    whzr_ref[...] = wh_ref[:, :2 * _D].astype(jnp.bfloat16)
    whh_ref[...] = wh_ref[:, 2 * _D:].astype(jnp.bfloat16)
    gx_ref[...] = (
        jnp.dot(e_ref[...], wx_ref[...], preferred_element_type=jnp.float32)
        + bx_ref[...]
    )

    def step(t, carry):
        h, acc = carry
        gxt = gx_ref[pl.ds(t * _B, _B), :]
        hb = h.astype(jnp.bfloat16)
        gh_zr = jnp.dot(hb, whzr_ref[...], preferred_element_type=jnp.float32)
        gh_h = jnp.dot(hb, whh_ref[...], preferred_element_type=jnp.float32)
        z = _sigmoid(gxt[:, :_D] + gh_zr[:, :_D] + bh_ref[:, :_D])
        r = _sigmoid(gxt[:, _D:2 * _D] + gh_zr[:, _D:] + bh_ref[:, _D:2 * _D])
        c = jnp.tanh(gxt[:, 2 * _D:] + r * (gh_h + bh_ref[:, 2 * _D:]))
        h2 = z * h + (1.0 - z) * c
        acc2 = acc + h2 * m_ref[t]
        return (h2, acc2)

    zeros = jnp.zeros((_B, _D), jnp.float32)
    _, acc = lax.fori_loop(0, _L, step, (zeros, zeros), unroll=8)
    cs_ref[...] = (acc / jnp.sqrt(jnp.float32(_D))).reshape(_B, 1, _D)


def _gru(e, wx, wh, bx, bh, m):
    return pl.pallas_call(
        _gru_body,
        out_shape=jax.ShapeDtypeStruct((_B, 1, _D), jnp.float32),
        scratch_shapes=[
            pltpu.VMEM((_B * _L, 3 * _D), jnp.float32),
            pltpu.VMEM((_D, 2 * _D), jnp.bfloat16),
            pltpu.VMEM((_D, _D), jnp.bfloat16),
        ],
    )(e, wx, wh, bx, bh, m)


def _score_body(keys_ref, x_ref, wg_ref, bg_ref, ws_ref, bs_ref, cs_ref,
                choice_ref, misc_ref, acc_ref):
    b = pl.program_id(0)
    h1t = jnp.tanh(
        lax.dot_general(wg_ref[...], x_ref[...], (((0,), (1,)), ((), ())),
                        preferred_element_type=jnp.float32)
        + bg_ref[...]
    )
    stt = lax.dot_general(ws_ref[...], h1t, (((0,), (0,)), ((), ())),
                          preferred_element_type=jnp.float32) + bs_ref[...]
    s = lax.dot_general(cs_ref[0], stt, (((1,), (0,)), ((), ())),
                        preferred_element_type=jnp.float32)
    smax = jnp.max(s)
    e = jnp.exp(s - smax)
    p = e / jnp.sum(e)
    choice_ref[...] = p.reshape(1, 1, _NPER)

    key_b = keys_ref[b]
    iota = lax.broadcasted_iota(jnp.int32, (1, _NPER), 1)
    p_at = jnp.sum(jnp.where(iota == key_b, p, 0.0))
    loss_b = -jnp.log(1e-6 + p_at)
    pmax = jnp.max(p)
    arg = jnp.min(jnp.where(p >= pmax, iota, jnp.int32(_NPER)))
    corr = jnp.where(arg == key_b, jnp.int32(1), jnp.int32(0))

    li = lax.broadcasted_iota(jnp.int32, (1, 128), 1)
    misc_ref[...] = jnp.where(li == 0, loss_b, 0.0).reshape(1, 1, 128)

    @pl.when(b == 0)
    def _():
        acc_ref[...] = jnp.zeros((1, 128), jnp.int32)

    acc_ref[...] += jnp.where(li == 0, corr, jnp.int32(0))


def _scores(keys, x, wg, bg_col, ws, bs_col, cs):
    return pl.pallas_call(
        _score_body,
        grid=(_B,),
        in_specs=[
            pl.BlockSpec(memory_space=pltpu.SMEM),
            pl.BlockSpec((_NPER, _D), lambda b: (b, 0)),
            pl.BlockSpec((_D, _DG), lambda b: (0, 0)),
            pl.BlockSpec((_DG, 1), lambda b: (0, 0)),
            pl.BlockSpec((_DG, _D), lambda b: (0, 0)),
            pl.BlockSpec((_D, 1), lambda b: (0, 0)),
            pl.BlockSpec((1, 1, _D), lambda b: (b, 0, 0)),
        ],
        out_specs=[
            pl.BlockSpec((1, 1, _NPER), lambda b: (b, 0, 0)),
            pl.BlockSpec((1, 1, 128), lambda b: (b, 0, 0)),
            pl.BlockSpec((1, 128), lambda b: (0, 0)),
        ],
        out_shape=[
            jax.ShapeDtypeStruct((_B, 1, _NPER), jnp.float32),
            jax.ShapeDtypeStruct((_B, 1, 128), jnp.float32),
            jax.ShapeDtypeStruct((1, 128), jnp.int32),
        ],
    )(keys, x, wg, bg_col, ws, bs_col, cs)


def kernel(x, comment_indexes, comment_masks, comment_keys, W_ggnn, b_ggnn,
           W_scale, b_scale, embed, Wx, Wh, bx, bh):
    m = comment_masks.astype(jnp.float32).T.reshape(_L, _B, 1)

    idx = comment_indexes.astype(jnp.int32).T.reshape(-1)
    e = _make_emb_gather()(embed, idx)
    cs = _gru(e, Wx, Wh, bx.reshape(1, -1), bh.reshape(1, -1), m)
    choice3, misc, accv = _scores(
        comment_keys.astype(jnp.int32), x, W_ggnn, b_ggnn.reshape(-1, 1),
        W_scale, b_scale.reshape(-1, 1), cs)

    choice = choice3.reshape(-1)
    loss_at = misc[:, 0, 0]
    acc = accv[0, 0]
    return (choice, loss_at, acc)
